# baseline (speedup 1.0000x reference)
import functools

import jax
import jax.numpy as jnp
from jax import lax
from jax.experimental import pallas as pl
from jax.experimental.pallas import tpu as pltpu
from jax.experimental.pallas import tpu_sc as plsc

N_NODES = 10000
D_FEAT = 128
N_EDGES = 320000

NC = 2
NS = 16
NW = NC * NS

CHUNK = 128
EPW = N_EDGES // NW
NCHUNK = 80
NHALF = NCHUNK // 2
EPW_PAD = NCHUNK * CHUNK
NP_ROWS = 10112
RPS = NP_ROWS // NS


_mesh = plsc.VectorSubcoreMesh(core_axis_name="c", subcore_axis_name="s")


@functools.partial(
    pl.kernel,
    mesh=_mesh,
    out_type=jax.ShapeDtypeStruct((NC, NP_ROWS, D_FEAT), jnp.float32),
    scratch_types=[
        pltpu.VMEM((NHALF, CHUNK), jnp.int32),
        pltpu.VMEM((NHALF, CHUNK), jnp.int32),
        pltpu.VMEM((CHUNK, D_FEAT), jnp.float32),
        pltpu.VMEM((CHUNK, D_FEAT), jnp.float32),
        pltpu.VMEM_SHARED((NP_ROWS, D_FEAT), jnp.float32),
        pltpu.SemaphoreType.DMA,
        pltpu.SemaphoreType.DMA,
        pltpu.SemaphoreType.DMA,
        pltpu.SemaphoreType.DMA,
    ],
)
def _sc_propagate(src_hbm, dst_hbm, x_hbm, out_hbm,
                  src_v, dst_v, rows0, rows1, acc_sh, g0, g1, s0, s1):
    rows = (rows0, rows1)
    gsem = (g0, g1)
    ssem = (s0, s1)

    cid = lax.axis_index("c")
    sid = lax.axis_index("s")
    wid = sid * NC + cid

    def gather_start(c, b):
        pltpu.async_copy(x_hbm.at[src_v.at[c]], rows[b], gsem[b])

    def gather_wait(c, b):
        pltpu.make_async_copy(x_hbm.at[src_v.at[c]], rows[b], gsem[b]).wait()

    def scatter_start(c, b):
        pltpu.async_copy(rows[b], acc_sh.at[dst_v.at[c]], ssem[b], add=True)

    def scatter_wait(c, b):
        pltpu.make_async_copy(rows[b], acc_sh.at[dst_v.at[c]], ssem[b]).wait()

    @pl.loop(0, CHUNK)
    def _(r):
        @pl.loop(0, D_FEAT, step=16)
        def _(c):
            rows0[r, pl.ds(c, 16)] = jnp.zeros((16,), jnp.float32)

    base = sid * RPS

    @pl.loop(0, 4)
    def _(k):
        pltpu.sync_copy(rows0, acc_sh.at[pl.ds(base + k * CHUNK, CHUNK)])

    pltpu.sync_copy(rows0.at[pl.ds(0, RPS - 4 * CHUNK)],
                    acc_sh.at[pl.ds(base + 4 * CHUNK, RPS - 4 * CHUNK)])

    plsc.subcore_barrier()

    for h in range(2):
        pltpu.sync_copy(src_hbm.at[wid].at[h], src_v)
        pltpu.sync_copy(dst_hbm.at[wid].at[h], dst_v)

        gather_start(0, 0)
        gather_wait(0, 0)
        scatter_start(0, 0)
        gather_start(1, 1)

        @pl.loop(1, NHALF - 1, step=2)
        def _(c):
            gather_wait(c, 1)
            scatter_start(c, 1)
            scatter_wait(c - 1, 0)
            gather_start(c + 1, 0)
            gather_wait(c + 1, 0)
            scatter_start(c + 1, 0)
            scatter_wait(c, 1)
            gather_start(c + 2, 1)

        gather_wait(NHALF - 1, 1)
        scatter_start(NHALF - 1, 1)
        scatter_wait(NHALF - 2, 0)
        scatter_wait(NHALF - 1, 1)

    plsc.subcore_barrier()

    pltpu.sync_copy(acc_sh.at[pl.ds(base, RPS)],
                    out_hbm.at[cid].at[pl.ds(base, RPS)])


def _combine_body(a_ref, b_ref, o_ref):
    o_ref[...] = a_ref[...] + b_ref[...]


def _combine(a, b):
    return pl.pallas_call(
        _combine_body,
        out_shape=jax.ShapeDtypeStruct((N_NODES, D_FEAT), jnp.float32),
        grid=(10,),
        in_specs=[pl.BlockSpec((N_NODES // 10, D_FEAT), lambda i: (i, 0)),
                  pl.BlockSpec((N_NODES // 10, D_FEAT), lambda i: (i, 0))],
        out_specs=pl.BlockSpec((N_NODES // 10, D_FEAT), lambda i: (i, 0)),
    )(a, b)


def kernel(edge_index, x):
    src = edge_index[0].reshape(NW, EPW)
    dst = edge_index[1].reshape(NW, EPW)
    pad = EPW_PAD - EPW
    src_p = jnp.pad(src, ((0, 0), (0, pad))).reshape(NW, 2, NHALF, CHUNK)
    dst_p = jnp.pad(dst, ((0, 0), (0, pad)),
                    constant_values=N_NODES).reshape(NW, 2, NHALF, CHUNK)
    partials = _sc_propagate(src_p, dst_p, x)
    return _combine(partials[0], partials[1])
